# Initial kernel scaffold; baseline (speedup 1.0000x reference)
import jax
import jax.numpy as jnp
from jax.experimental import pallas as pl


def kernel(node_feats, edge_feats, params, edge_index, graph_ids):
    raise NotImplementedError("write your pallas kernel here")


# speedup vs baseline: 1.0987x; 1.0987x over previous
import jax
import jax.numpy as jnp
from jax.experimental import pallas as pl

H = 200
G = 256


def _leaky(x):
    return jnp.where(x >= 0, x, 0.01 * x)


def _elu(x):
    return jnp.where(x > 0, x, jnp.exp(jnp.minimum(x, 0.0)) - 1.0)


def _mm_rows(x, Wt, b, act, block):
    M, K = x.shape
    O = Wt.shape[1]

    def kern(x_ref, w_ref, b_ref, o_ref):
        acc = jnp.dot(x_ref[...], w_ref[...], preferred_element_type=jnp.float32)
        o_ref[...] = act(acc + b_ref[...])

    return pl.pallas_call(
        kern,
        grid=(M // block,),
        in_specs=[
            pl.BlockSpec((block, K), lambda i: (i, 0)),
            pl.BlockSpec((K, O), lambda i: (0, 0)),
            pl.BlockSpec((1, O), lambda i: (0, 0)),
        ],
        out_specs=pl.BlockSpec((block, O), lambda i: (i, 0)),
        out_shape=jax.ShapeDtypeStruct((M, O), jnp.float32),
    )(x, Wt, b.reshape(1, O))


def _edge1(nf_src, ef, hv_dst, W1a, W1b, b1, W2a, W2b, b2, Wet, bet, block):
    E = nf_src.shape[0]

    def kern(ns, efr, hd, w1a, w1b, b1r, w2a, w2b, b2r, wet, betr, lo, eo):
        he1 = _leaky(
            jnp.dot(ns[...], w1a[...], preferred_element_type=jnp.float32)
            + jnp.dot(efr[...], w1b[...], preferred_element_type=jnp.float32)
            + b1r[...]
        )
        lo[...] = _leaky(
            jnp.dot(hd[...], w2a[...], preferred_element_type=jnp.float32)
            + jnp.dot(he1, w2b[...], preferred_element_type=jnp.float32)
            + b2r[...]
        )
        eo[...] = (
            jnp.dot(he1, wet[...], preferred_element_type=jnp.float32) + betr[...]
        )

    full = lambda r, c: pl.BlockSpec((r, c), lambda i: (0, 0))
    return pl.pallas_call(
        kern,
        grid=(E // block,),
        in_specs=[
            pl.BlockSpec((block, nf_src.shape[1]), lambda i: (i, 0)),
            pl.BlockSpec((block, ef.shape[1]), lambda i: (i, 0)),
            pl.BlockSpec((block, H), lambda i: (i, 0)),
            full(nf_src.shape[1], H),
            full(ef.shape[1], H),
            full(1, H),
            full(H, 1),
            full(H, 1),
            full(1, 1),
            full(H, H),
            full(1, H),
        ],
        out_specs=[
            pl.BlockSpec((block, 1), lambda i: (i, 0)),
            pl.BlockSpec((block, H), lambda i: (i, 0)),
        ],
        out_shape=[
            jax.ShapeDtypeStruct((E, 1), jnp.float32),
            jax.ShapeDtypeStruct((E, H), jnp.float32),
        ],
    )(nf_src, ef, hv_dst, W1a, W1b, b1, W2a, W2b, b2, Wet, bet)


def _edge_logits(xa, xb, Wa, Wb, b, block, relu_a=False):
    E = xa.shape[0]

    def kern(ar, br, wa, wb, bb, lo):
        a = ar[...]
        if relu_a:
            a = jnp.maximum(a, 0.0)
        lo[...] = _leaky(
            jnp.dot(a, wa[...], preferred_element_type=jnp.float32)
            + jnp.dot(br[...], wb[...], preferred_element_type=jnp.float32)
            + bb[...]
        )

    full = lambda r, c: pl.BlockSpec((r, c), lambda i: (0, 0))
    return pl.pallas_call(
        kern,
        grid=(E // block,),
        in_specs=[
            pl.BlockSpec((block, H), lambda i: (i, 0)),
            pl.BlockSpec((block, H), lambda i: (i, 0)),
            full(H, 1),
            full(H, 1),
            full(1, 1),
        ],
        out_specs=pl.BlockSpec((block, 1), lambda i: (i, 0)),
        out_shape=jax.ShapeDtypeStruct((E, 1), jnp.float32),
    )(xa, xb, Wa, Wb, b)


def _gru(ctx_raw, h, ws, block):
    M = ctx_raw.shape[0]

    def kern(xr, hr, wir, wiz, win, whr, whz, whn, bir, biz, bin_, bhr, bhz, bhn, o):
        x = _elu(xr[...])
        hh = hr[...]
        r = jax.nn.sigmoid(
            jnp.dot(x, wir[...], preferred_element_type=jnp.float32)
            + bir[...]
            + jnp.dot(hh, whr[...], preferred_element_type=jnp.float32)
            + bhr[...]
        )
        z = jax.nn.sigmoid(
            jnp.dot(x, wiz[...], preferred_element_type=jnp.float32)
            + biz[...]
            + jnp.dot(hh, whz[...], preferred_element_type=jnp.float32)
            + bhz[...]
        )
        n = jnp.tanh(
            jnp.dot(x, win[...], preferred_element_type=jnp.float32)
            + bin_[...]
            + r
            * (
                jnp.dot(hh, whn[...], preferred_element_type=jnp.float32)
                + bhn[...]
            )
        )
        o[...] = jnp.maximum((1.0 - z) * n + z * hh, 0.0)

    full = lambda r, c: pl.BlockSpec((r, c), lambda i: (0, 0))
    row = pl.BlockSpec((block, H), lambda i: (i, 0))
    return pl.pallas_call(
        kern,
        grid=(M // block,),
        in_specs=[row, row] + [full(H, H)] * 6 + [full(1, H)] * 6,
        out_specs=row,
        out_shape=jax.ShapeDtypeStruct((M, H), jnp.float32),
    )(ctx_raw, h, *ws)


def _split_gru(Wih, Whh, bih, bhh):
    return (
        Wih[:H].T, Wih[H : 2 * H].T, Wih[2 * H :].T,
        Whh[:H].T, Whh[H : 2 * H].T, Whh[2 * H :].T,
        bih[:H].reshape(1, H), bih[H : 2 * H].reshape(1, H), bih[2 * H :].reshape(1, H),
        bhh[:H].reshape(1, H), bhh[H : 2 * H].reshape(1, H), bhh[2 * H :].reshape(1, H),
    )


def _seg_softmax(logits, seg, num_segments):
    m = jax.ops.segment_max(logits, seg, num_segments=num_segments)
    m = jnp.where(jnp.isfinite(m), m, 0.0)
    ex = jnp.exp(logits - m[seg])
    s = jax.ops.segment_sum(ex, seg, num_segments=num_segments)
    return ex / (s[seg] + 1e-12)


def kernel(node_feats, edge_feats, params, edge_index, graph_ids):
    p = params
    src = edge_index[0]
    dst = edge_index[1]
    N = node_feats.shape[0]
    BE = 2000
    BN = 2000

    hv_new = _mm_rows(node_feats, p["W_pn"].T, p["b_pn"], _leaky, BN)
    nf_src = node_feats[src]
    hv_dst = hv_new[dst]
    DN = node_feats.shape[1]
    W1 = p["W_pe1"].T
    W2 = p["W_pe2"].T
    logits, et = _edge1(
        nf_src, edge_feats, hv_dst,
        W1[:DN], W1[DN:], p["b_pe1"].reshape(1, H),
        W2[:H], W2[H:], p["b_pe2"].reshape(1, 1),
        p["W_et"].T, p["b_et"].reshape(1, H), BE,
    )
    a = _seg_softmax(logits[:, 0], dst, N)
    ctx_raw = jax.ops.segment_sum(a[:, None] * et, dst, num_segments=N)
    h = _gru(ctx_raw, hv_new, _split_gru(p["Wih1"], p["Whh1"], p["bih1"], p["bhh1"]), BN)

    Wl = p["W_pe"].T
    logits2 = _edge_logits(h[dst], h[src], Wl[:H], Wl[H:], p["b_pe"].reshape(1, 1), BE)
    a2 = _seg_softmax(logits2[:, 0], dst, N)
    hv_proj = _mm_rows(h, p["W_pn2"].T, p["b_pn2"], lambda x: x, BN)
    ctx2 = jax.ops.segment_sum(hv_proj[src] * a2[:, None], dst, num_segments=N)
    h = _gru(ctx2, h, _split_gru(p["Wih2"], p["Whh2"], p["bih2"], p["bhh2"]), BN)

    g_feats = jax.ops.segment_sum(h, graph_ids, num_segments=G)
    for t in range(2):
        Wz = p["Wz%d" % t].T
        z = _edge_logits(g_feats[graph_ids], h, Wz[:H], Wz[H:], p["bz%d" % t].reshape(1, 1), BN, relu_a=True)
        a3 = _seg_softmax(z[:, 0], graph_ids, G)
        hvp = _mm_rows(h, p["Wp%d" % t].T, p["bp%d" % t], lambda x: x, BN)
        g_repr = jax.ops.segment_sum(hvp * a3[:, None], graph_ids, num_segments=G)
        g_feats = _gru(
            g_repr, g_feats,
            _split_gru(p["Wihr%d" % t], p["Whhr%d" % t], p["bihr%d" % t], p["bhhr%d" % t]),
            G,
        )
    return g_feats
